# Initial kernel scaffold; baseline (speedup 1.0000x reference)
import jax
import jax.numpy as jnp
from jax.experimental import pallas as pl


def kernel(train_x, item_w, bias_user_w, bias_item_w, bias, user_taste, user_attnd):
    raise NotImplementedError("write your pallas kernel here")


# speedup vs baseline: 6.8290x; 6.8290x over previous
import functools

import jax
import jax.numpy as jnp
from jax import lax
from jax.experimental import pallas as pl
from jax.experimental.pallas import tpu as pltpu
from jax.experimental.pallas import tpu_sc as plsc

NC = 2
NS = 16
L = 16
NW = NC * NS
K = 32
C = 4
KC = K * C
CH = 128


def _mf_body(tx, itw, buw, biw, bias, utw, uaw, out,
             txv, uidv, iidv, uav, utv, itv, buv, biv, outv, biasv, sem):
    b = out.shape[0]
    bpw = b // NW
    nchunk = bpw // CH
    wid = lax.axis_index("s") * NC + lax.axis_index("c")
    base = wid * bpw

    pltpu.sync_copy(bias, biasv.at[pl.ds(0, 1)])
    b0 = biasv[...][0]
    zeros16 = jnp.zeros((L,), jnp.int32)
    ones16 = jnp.ones((L,), jnp.int32)
    lane = lax.iota(jnp.int32, L)

    for ci in range(nchunk):
        cbase = base + ci * CH
        pltpu.sync_copy(tx.at[pl.ds(cbase, CH)], txv)

        def extract(j, _):
            e = j * L + lane
            uidv[pl.ds(j * L, L)] = plsc.load_gather(txv, [e, zeros16])
            iidv[pl.ds(j * L, L)] = plsc.load_gather(txv, [e, ones16])
            return 0
        lax.fori_loop(0, CH // L, extract, 0)

        cps = [pltpu.async_copy(uaw.at[uidv], uav, sem),
               pltpu.async_copy(utw.at[uidv], utv, sem),
               pltpu.async_copy(itw.at[iidv], itv, sem),
               pltpu.async_copy(buw.at[uidv], buv, sem),
               pltpu.async_copy(biw.at[iidv], biv, sem)]
        for cp in cps:
            cp.wait()

        def group(g, _):
            e = g * L + lane

            def kbody(k, carry):
                z0, z1, z2, z3, w0, w1, w2, w3 = carry
                kk = jnp.full((L,), k, jnp.int32)
                vi = plsc.load_gather(itv, [e, kk])
                c0 = 4 * k
                f0 = jnp.full((L,), c0, jnp.int32)
                f1 = jnp.full((L,), c0 + 1, jnp.int32)
                f2 = jnp.full((L,), c0 + 2, jnp.int32)
                f3 = jnp.full((L,), c0 + 3, jnp.int32)
                uts = (plsc.load_gather(utv, [e, f0])
                       + plsc.load_gather(utv, [e, f1])
                       + plsc.load_gather(utv, [e, f2])
                       + plsc.load_gather(utv, [e, f3]))
                p = vi * uts
                a0 = jnp.exp(plsc.load_gather(uav, [e, f0]) * vi)
                a1 = jnp.exp(plsc.load_gather(uav, [e, f1]) * vi)
                a2 = jnp.exp(plsc.load_gather(uav, [e, f2]) * vi)
                a3 = jnp.exp(plsc.load_gather(uav, [e, f3]) * vi)
                return (z0 + a0, z1 + a1, z2 + a2, z3 + a3,
                        w0 + p * a0, w1 + p * a1, w2 + p * a2, w3 + p * a3)

            zf = jnp.zeros((L,), jnp.float32)
            z0, z1, z2, z3, w0, w1, w2, w3 = lax.fori_loop(
                0, K, kbody, (zf, zf, zf, zf, zf, zf, zf, zf))
            dot = w0 / z0 + w1 / z1 + w2 / z2 + w3 / z3
            bu = buv[pl.ds(g * L, L)]
            bi_ = biv[pl.ds(g * L, L)]
            outv[pl.ds(g * L, L)] = dot + b0 + bu + bi_
            return 0
        lax.fori_loop(0, CH // L, group, 0)

        pltpu.sync_copy(outv, out.at[pl.ds(cbase, CH)])


def kernel(train_x, item_w, bias_user_w, bias_item_w, bias, user_taste, user_attnd):
    b = train_x.shape[0]
    assert b % (NW * CH) == 0
    tx = train_x.astype(jnp.int32)
    ut2 = user_taste.reshape(user_taste.shape[0], KC)
    ua2 = user_attnd.reshape(user_attnd.shape[0], KC)
    bu1 = bias_user_w.reshape(-1)
    bi1 = bias_item_w.reshape(-1)
    mesh = plsc.VectorSubcoreMesh(core_axis_name="c", subcore_axis_name="s")
    kfn = pl.kernel(
        _mf_body,
        mesh=mesh,
        compiler_params=pltpu.CompilerParams(
            needs_layout_passes=False, use_tc_tiling_on_sc=False),
        out_type=jax.ShapeDtypeStruct((b,), jnp.float32),
        scratch_types=[
            pltpu.VMEM((CH, 2), jnp.int32),
            pltpu.VMEM((CH,), jnp.int32),
            pltpu.VMEM((CH,), jnp.int32),
            pltpu.VMEM((CH, KC), jnp.float32),
            pltpu.VMEM((CH, KC), jnp.float32),
            pltpu.VMEM((CH, K), jnp.float32),
            pltpu.VMEM((CH,), jnp.float32),
            pltpu.VMEM((CH,), jnp.float32),
            pltpu.VMEM((CH,), jnp.float32),
            pltpu.VMEM((L,), jnp.float32),
            pltpu.SemaphoreType.DMA,
        ],
    )
    return kfn(tx, item_w, bu1, bi1, bias, ut2, ua2)
